# baseline (speedup 1.0000x reference)
import jax
import jax.numpy as jnp
from jax.experimental import pallas as pl
from jax.experimental.pallas import tpu as pltpu

_N = 1_000_000
_CBLK = 65_536
_GRID = -(-_N // _CBLK)


def _norm_sum_body(u_ref, v_ref, o_ref, acc_ref):
    step = pl.program_id(0)

    @pl.when(step == 0)
    def _init():
        acc_ref[...] = jnp.zeros_like(acc_ref)

    ones_row = jnp.ones((1, 8), jnp.float32)
    col = jax.lax.broadcasted_iota(jnp.int32, (1, _CBLK), 1) + step * _CBLK
    valid = col < _N

    def block_norms(x):
        x2 = x * x
        z = x2[0:8, :] + x2[8:16, :] + x2[16:24, :] + x2[24:32, :]
        n2 = jax.lax.dot_general(
            ones_row, z, (((1,), (0,)), ((), ())),
            preferred_element_type=jnp.float32)
        return jnp.where(valid, jnp.sqrt(n2), 0.0)

    acc_ref[0:1, :] += block_norms(u_ref[...]) + block_norms(v_ref[...])

    @pl.when(step == _GRID - 1)
    def _fin():
        o_ref[0, 0] = jnp.sum(acc_ref[0:1, :])


def kernel(user_embedding, item_embedding):
    ut = user_embedding.T
    vt = item_embedding.T
    out = pl.pallas_call(
        _norm_sum_body,
        grid=(_GRID,),
        in_specs=[
            pl.BlockSpec((32, _CBLK), lambda i: (0, i)),
            pl.BlockSpec((32, _CBLK), lambda i: (0, i)),
        ],
        out_specs=pl.BlockSpec(memory_space=pltpu.SMEM),
        out_shape=jax.ShapeDtypeStruct((1, 1), jnp.float32),
        scratch_shapes=[pltpu.VMEM((8, _CBLK), jnp.float32)],
    )(ut, vt)
    return out[0, 0]


# speedup vs baseline: 1.0194x; 1.0194x over previous
import jax
import jax.numpy as jnp
from jax.experimental import pallas as pl
from jax.experimental.pallas import tpu as pltpu

_N = 1_000_000
_CBLK = 25_600
_NBLK = -(-_N // _CBLK)
_SPT = 2
_GRID = _NBLK // _SPT


def _norm_sum_body(u0_ref, u1_ref, v0_ref, v1_ref, o_ref, acc_ref):
    step = pl.program_id(0)

    @pl.when(step == 0)
    def _init():
        acc_ref[...] = jnp.zeros_like(acc_ref)

    ones_row = jnp.ones((1, 8), jnp.float32)
    lane = jax.lax.broadcasted_iota(jnp.int32, (1, _CBLK), 1)

    def block_norms(x, blk_idx):
        col = lane + blk_idx * _CBLK
        x2 = x * x
        z = x2[0:8, :] + x2[8:16, :] + x2[16:24, :] + x2[24:32, :]
        n2 = jax.lax.dot_general(
            ones_row, z, (((1,), (0,)), ((), ())),
            preferred_element_type=jnp.float32)
        return jnp.where(col < _N, jnp.sqrt(n2), 0.0)

    acc_ref[0:1, :] += (
        block_norms(u0_ref[...], step)
        + block_norms(u1_ref[...], _GRID + step)
        + block_norms(v0_ref[...], step)
        + block_norms(v1_ref[...], _GRID + step)
    )

    @pl.when(step == _GRID - 1)
    def _fin():
        o_ref[0, 0] = jnp.sum(acc_ref[0:1, :])


def kernel(user_embedding, item_embedding):
    ut = user_embedding.T
    vt = item_embedding.T
    spec0 = pl.BlockSpec((32, _CBLK), lambda i: (0, i))
    spec1 = pl.BlockSpec((32, _CBLK), lambda i: (0, _GRID + i))
    out = pl.pallas_call(
        _norm_sum_body,
        grid=(_GRID,),
        in_specs=[spec0, spec1, spec0, spec1],
        out_specs=pl.BlockSpec(memory_space=pltpu.SMEM),
        out_shape=jax.ShapeDtypeStruct((1, 1), jnp.float32),
        scratch_shapes=[pltpu.VMEM((8, _CBLK), jnp.float32)],
    )(ut, ut, vt, vt)
    return out[0, 0]
